# Initial kernel scaffold; baseline (speedup 1.0000x reference)
import jax
import jax.numpy as jnp
from jax.experimental import pallas as pl


def kernel(x, entity_embeddings, relationship_embeddings):
    raise NotImplementedError("write your pallas kernel here")


# speedup vs baseline: 1.7600x; 1.7600x over previous
import functools

import jax
import jax.numpy as jnp
from jax import lax
from jax.experimental import pallas as pl
from jax.experimental.pallas import tpu as pltpu
from jax.experimental.pallas import tpu_sc as plsc

_N = 100
_D = 50
_B = 16384
_NC = 2
_NS = 16
_NW = _NC * _NS
_L = 16
_BPW = _B // _NW
_NPAD = 112


def _sqrt16(x):
    xs = x + 1e-30
    seed = plsc.bitcast(
        jnp.int32(0x5F3759DF) - (plsc.bitcast(xs, jnp.int32) >> 1), jnp.float32)
    y = seed
    for _ in range(3):
        y = y * (1.5 - 0.5 * xs * y * y)
    return xs * y


def _sc_body(x_hbm, tab_hbm, out_hbm, x_v, tab_v, norms_v, acc_v):
    wid = lax.axis_index("s") * _NC + lax.axis_index("c")
    base = wid * (_BPW * 3)
    pltpu.sync_copy(x_hbm.at[pl.ds(base, _BPW * 3)], x_v)
    pltpu.sync_copy(tab_hbm, tab_v)

    lanes = lax.iota(jnp.int32, 16)

    for g in range(_NPAD // _L):
        rows = jnp.minimum(lanes + g * _L, _N - 1)

        def nbody(d, a):
            v = plsc.load_gather(tab_v, [rows * _D + d])
            return a + v * v

        sq = lax.fori_loop(0, _D, nbody, jnp.zeros((_L,), jnp.float32))
        norms_v[pl.ds(g * _L, _L)] = _sqrt16(sq)

    def body(i, a):
        xi = plsc.load_gather(x_v, [(i * _L + lanes) * 3 + 1])
        nv = plsc.load_gather(norms_v, [xi])
        return a + nv

    acc = lax.fori_loop(0, _BPW // _L, body, jnp.zeros((_L,), jnp.float32))
    acc_v[...] = acc
    pltpu.sync_copy(acc_v, out_hbm.at[wid])


_sc_partials = functools.partial(
    pl.kernel,
    mesh=plsc.VectorSubcoreMesh(core_axis_name="c", subcore_axis_name="s"),
    out_type=jax.ShapeDtypeStruct((_NW, _L), jnp.float32),
    compiler_params=pltpu.CompilerParams(needs_layout_passes=False),
    scratch_types=[
        pltpu.VMEM((_BPW * 3,), jnp.int32),
        pltpu.VMEM((_N * _D,), jnp.float32),
        pltpu.VMEM((_NPAD,), jnp.float32),
        pltpu.VMEM((_L,), jnp.float32),
    ],
)(_sc_body)


def _reduce_body(p_ref, o_ref):
    o_ref[...] = jnp.sum(p_ref[...], keepdims=True) * (1.0 / _B)


def kernel(x, entity_embeddings, relationship_embeddings):
    del relationship_embeddings
    xf = x.astype(jnp.int32).reshape(-1)
    tf = entity_embeddings.reshape(-1)
    partials = _sc_partials(xf, tf)
    loss = pl.pallas_call(
        _reduce_body,
        out_shape=jax.ShapeDtypeStruct((1, 1), jnp.float32),
    )(partials)
    return loss[0, 0]


# speedup vs baseline: 2.6396x; 1.4998x over previous
import functools

import jax
import jax.numpy as jnp
from jax import lax
from jax.experimental import pallas as pl
from jax.experimental.pallas import tpu as pltpu
from jax.experimental.pallas import tpu_sc as plsc

_N = 100
_D = 50
_B = 16384
_NC = 2
_NS = 16
_NW = _NC * _NS
_L = 16
_BPW = _B // _NW
_NPAD = 112


def _sqrt16(x):
    xs = x + 1e-30
    seed = plsc.bitcast(
        jnp.int32(0x5F3759DF) - (plsc.bitcast(xs, jnp.int32) >> 1), jnp.float32)
    y = seed
    for _ in range(3):
        y = y * (1.5 - 0.5 * xs * y * y)
    return xs * y


def _sc_body(idx_hbm, tab_hbm, out_hbm, idx_v, tab_v, norms_v, acc_v):
    wid = lax.axis_index("s") * _NC + lax.axis_index("c")
    base = wid * _BPW
    pltpu.sync_copy(idx_hbm.at[pl.ds(base, _BPW)], idx_v)
    pltpu.sync_copy(tab_hbm, tab_v)

    lanes = lax.iota(jnp.int32, 16)

    zero = jnp.zeros((_L,), jnp.float32)
    for g in range(_NPAD // _L):
        rows = jnp.minimum(lanes + g * _L, _N - 1)
        rowbase = rows * _D
        accs = [zero, zero, zero, zero]
        for d in range(_D):
            v = plsc.load_gather(tab_v, [rowbase + d])
            accs[d % 4] = accs[d % 4] + v * v
        sq = (accs[0] + accs[1]) + (accs[2] + accs[3])
        norms_v[pl.ds(g * _L, _L)] = _sqrt16(sq)

    accs = [zero, zero, zero, zero]
    for i in range(_BPW // _L):
        xi = idx_v[pl.ds(i * _L, _L)]
        nv = plsc.load_gather(norms_v, [xi])
        accs[i % 4] = accs[i % 4] + nv
    acc_v[...] = (accs[0] + accs[1]) + (accs[2] + accs[3])
    pltpu.sync_copy(acc_v, out_hbm.at[wid])


_sc_partials = functools.partial(
    pl.kernel,
    mesh=plsc.VectorSubcoreMesh(core_axis_name="c", subcore_axis_name="s"),
    out_type=jax.ShapeDtypeStruct((_NW, _L), jnp.float32),
    compiler_params=pltpu.CompilerParams(needs_layout_passes=False),
    scratch_types=[
        pltpu.VMEM((_BPW,), jnp.int32),
        pltpu.VMEM((_N * _D,), jnp.float32),
        pltpu.VMEM((_NPAD,), jnp.float32),
        pltpu.VMEM((_L,), jnp.float32),
    ],
)(_sc_body)


def _reduce_body(p_ref, o_ref):
    o_ref[...] = jnp.sum(p_ref[...], keepdims=True) * (1.0 / _B)


def kernel(x, entity_embeddings, relationship_embeddings):
    del relationship_embeddings
    idx = x[:, 1].astype(jnp.int32)
    tf = entity_embeddings.reshape(-1)
    partials = _sc_partials(idx, tf)
    loss = pl.pallas_call(
        _reduce_body,
        out_shape=jax.ShapeDtypeStruct((1, 1), jnp.float32),
    )(partials)
    return loss[0, 0]
